# Initial kernel scaffold; baseline (speedup 1.0000x reference)
import jax
import jax.numpy as jnp
from jax.experimental import pallas as pl


def kernel(data, adj, W1, b1, W2, b2):
    raise NotImplementedError("write your pallas kernel here")


# speedup vs baseline: 19.3681x; 19.3681x over previous
import functools

import jax
import jax.numpy as jnp
from jax import lax
from jax.experimental import pallas as pl
from jax.experimental.pallas import tpu as pltpu
from jax.experimental.pallas import tpu_sc as plsc

N_NODES = 10000
N_EDGES = 320000
N_FEAT = 128
HIDN = 16
N_CLASSES = 40

NP = 10240
PAD = N_NODES
NC, NS, LANES = 2, 16, 16
NW = NC * NS
CHUNK = 128
NCHUNK_T = 79
EPT = NCHUNK_T * CHUNK
EP = NW * EPT
C2 = 48
ROWS_PER_TILE = NP // NS
BLK = 1024


def _mesh():
    return plsc.VectorSubcoreMesh(
        core_axis_name="c", subcore_axis_name="s", num_cores=NC, num_subcores=NS
    )


def _sc_deg(dst_p):
    def body(dst_hbm, out_hbm, dst_v, deg_v):
        c = lax.axis_index("c")
        s = lax.axis_index("s")
        w = s * NC + c
        zero16 = jnp.zeros((LANES,), jnp.float32)
        one16 = jnp.ones((LANES,), jnp.float32)

        def zloop(i, _):
            deg_v[pl.ds(i * LANES, LANES)] = zero16
            return None

        lax.fori_loop(0, NP // LANES, zloop, None)
        pltpu.sync_copy(dst_hbm.at[pl.ds(w * EPT, EPT)], dst_v)

        def eloop(i, _):
            idx = dst_v[pl.ds(i * LANES, LANES)]
            plsc.addupdate_scatter(deg_v, [idx], one16)
            return None

        lax.fori_loop(0, EPT // LANES, eloop, None)
        pltpu.sync_copy(deg_v, out_hbm.at[w])

    fn = pl.kernel(
        body,
        out_type=jax.ShapeDtypeStruct((NW, NP), jnp.float32),
        mesh=_mesh(),
        scratch_types=[
            pltpu.VMEM((EPT,), jnp.int32),
            pltpu.VMEM((NP,), jnp.float32),
        ],
        compiler_params=pltpu.CompilerParams(needs_layout_passes=False),
    )
    return fn(dst_p)


def _sc_scatter(g, src2, dst2, d):
    def body(g_hbm, src_hbm, dst_hbm, out_hbm, idx_s, idx_d, rows, zbuf, sem, acc):
        c = lax.axis_index("c")
        s = lax.axis_index("s")
        w = s * NC + c
        zero16 = jnp.zeros((LANES,), jnp.float32)

        def zrow(i, _):
            for j in range(d // LANES):
                zbuf[i, pl.ds(j * LANES, LANES)] = zero16
            return None

        lax.fori_loop(0, CHUNK, zrow, None)

        def zacc(k, _):
            pltpu.sync_copy(zbuf, acc.at[pl.ds(s * ROWS_PER_TILE + k * CHUNK, CHUNK)])
            return None

        lax.fori_loop(0, ROWS_PER_TILE // CHUNK, zacc, None)
        plsc.subcore_barrier()

        def step(gi, _):
            row = w * NCHUNK_T + gi
            pltpu.sync_copy(src_hbm.at[row], idx_s)
            pltpu.sync_copy(dst_hbm.at[row], idx_d.at[0])
            pltpu.async_copy(g_hbm.at[idx_s], rows, sem).wait()
            pltpu.sync_copy(rows, acc.at[idx_d.at[0]], add=True)
            return None

        lax.fori_loop(0, NCHUNK_T, step, None)
        plsc.subcore_barrier()

        def flush(k, _):
            a = s * ROWS_PER_TILE + k * CHUNK
            pltpu.sync_copy(acc.at[pl.ds(a, CHUNK)], out_hbm.at[c, pl.ds(a, CHUNK)])
            return None

        lax.fori_loop(0, ROWS_PER_TILE // CHUNK, flush, None)

    fn = pl.kernel(
        body,
        out_type=jax.ShapeDtypeStruct((NC, NP, d), jnp.float32),
        mesh=_mesh(),
        scratch_types=[
            pltpu.VMEM((CHUNK,), jnp.int32),
            pltpu.VMEM((1, CHUNK), jnp.int32),
            pltpu.VMEM((CHUNK, d), jnp.float32),
            pltpu.VMEM((CHUNK, d), jnp.float32),
            pltpu.SemaphoreType.DMA,
            pltpu.VMEM_SHARED((NP, d), jnp.float32),
        ],
        compiler_params=pltpu.CompilerParams(
            needs_layout_passes=False, use_tc_tiling_on_sc=False
        ),
    )
    return fn(g, src2, dst2)


def _tc_pre_body(degp_ref, data_ref, w1_ref, h1_ref, g1_ref, dis_ref):
    deg = jnp.sum(degp_ref[...], axis=0) + 1.0
    dis = lax.rsqrt(deg)[:, None]
    h = jnp.dot(data_ref[...], w1_ref[...], preferred_element_type=jnp.float32)
    h1_ref[...] = h
    g1_ref[...] = h * dis
    dis_ref[...] = dis


def _tc_pre(degp, data_p, W1):
    grid = NP // BLK
    return pl.pallas_call(
        _tc_pre_body,
        grid=(grid,),
        in_specs=[
            pl.BlockSpec((NW, BLK), lambda i: (0, i)),
            pl.BlockSpec((BLK, N_FEAT), lambda i: (i, 0)),
            pl.BlockSpec((N_FEAT, HIDN), lambda i: (0, 0)),
        ],
        out_specs=[
            pl.BlockSpec((BLK, HIDN), lambda i: (i, 0)),
            pl.BlockSpec((BLK, HIDN), lambda i: (i, 0)),
            pl.BlockSpec((BLK, 1), lambda i: (i, 0)),
        ],
        out_shape=[
            jax.ShapeDtypeStruct((NP, HIDN), jnp.float32),
            jax.ShapeDtypeStruct((NP, HIDN), jnp.float32),
            jax.ShapeDtypeStruct((NP, 1), jnp.float32),
        ],
    )(degp, data_p, W1)


def _tc_mid_body(p1_ref, h1_ref, dis_ref, w2_ref, b1_ref, g2_ref, h2_ref):
    i = pl.program_id(0)
    dis = dis_ref[...]
    acc = p1_ref[0] + p1_ref[1]
    x1 = dis * acc + (dis * dis) * h1_ref[...] + b1_ref[...]
    x1 = jnp.maximum(x1, 0.0)
    rows = i * BLK + lax.broadcasted_iota(jnp.int32, (BLK, 1), 0)
    x1 = jnp.where(rows < N_NODES, x1, 0.0)
    h2 = jnp.dot(x1, w2_ref[...], preferred_element_type=jnp.float32)
    h2_ref[...] = h2
    g2_ref[...] = h2 * dis


def _tc_mid(p1, h1, dis, w2p, b1r):
    grid = NP // BLK
    return pl.pallas_call(
        _tc_mid_body,
        grid=(grid,),
        in_specs=[
            pl.BlockSpec((NC, BLK, HIDN), lambda i: (0, i, 0)),
            pl.BlockSpec((BLK, HIDN), lambda i: (i, 0)),
            pl.BlockSpec((BLK, 1), lambda i: (i, 0)),
            pl.BlockSpec((HIDN, C2), lambda i: (0, 0)),
            pl.BlockSpec((1, HIDN), lambda i: (0, 0)),
        ],
        out_specs=[
            pl.BlockSpec((BLK, C2), lambda i: (i, 0)),
            pl.BlockSpec((BLK, C2), lambda i: (i, 0)),
        ],
        out_shape=[
            jax.ShapeDtypeStruct((NP, C2), jnp.float32),
            jax.ShapeDtypeStruct((NP, C2), jnp.float32),
        ],
    )(p1, h1, dis, w2p, b1r)


def _tc_post_body(p2_ref, h2_ref, dis_ref, b2_ref, out_ref):
    dis = dis_ref[...]
    acc = p2_ref[0] + p2_ref[1]
    out_ref[...] = dis * acc + (dis * dis) * h2_ref[...] + b2_ref[...]


def _tc_post(p2, h2, dis, b2r):
    grid = NP // BLK
    return pl.pallas_call(
        _tc_post_body,
        grid=(grid,),
        in_specs=[
            pl.BlockSpec((NC, BLK, C2), lambda i: (0, i, 0)),
            pl.BlockSpec((BLK, C2), lambda i: (i, 0)),
            pl.BlockSpec((BLK, 1), lambda i: (i, 0)),
            pl.BlockSpec((1, C2), lambda i: (0, 0)),
        ],
        out_specs=pl.BlockSpec((BLK, C2), lambda i: (i, 0)),
        out_shape=jax.ShapeDtypeStruct((NP, C2), jnp.float32),
    )(p2, h2, dis, b2r)


def kernel(data, adj, W1, b1, W2, b2):
    src = adj[0].astype(jnp.int32)
    dst = adj[1].astype(jnp.int32)
    pad = jnp.full((EP - N_EDGES,), PAD, jnp.int32)
    src_p = jnp.concatenate([src, pad])
    dst_p = jnp.concatenate([dst, pad])
    src2 = src_p.reshape(EP // CHUNK, CHUNK)
    dst2 = dst_p.reshape(EP // CHUNK, CHUNK)
    data_p = jnp.zeros((NP, N_FEAT), jnp.float32).at[:N_NODES].set(data)
    w2p = jnp.zeros((HIDN, C2), jnp.float32).at[:, :N_CLASSES].set(W2)
    b1r = b1.reshape(1, HIDN)
    b2r = jnp.zeros((1, C2), jnp.float32).at[0, :N_CLASSES].set(b2)

    degp = _sc_deg(dst_p)
    h1, g1, dis = _tc_pre(degp, data_p, W1)
    p1 = _sc_scatter(g1, src2, dst2, HIDN)
    g2, h2 = _tc_mid(p1, h1, dis, w2p, b1r)
    p2 = _sc_scatter(g2, src2, dst2, C2)
    outp = _tc_post(p2, h2, dis, b2r)
    return outp[:N_NODES, :N_CLASSES]


# speedup vs baseline: 23.9543x; 1.2368x over previous
import functools

import jax
import jax.numpy as jnp
from jax import lax
from jax.experimental import pallas as pl
from jax.experimental.pallas import tpu as pltpu
from jax.experimental.pallas import tpu_sc as plsc

N_NODES = 10000
N_EDGES = 320000
N_FEAT = 128
HIDN = 16
N_CLASSES = 40

NP = 10240
PAD = N_NODES
NC, NS, LANES = 2, 16, 16
NW = NC * NS
CHUNK = 128
NCHUNK_T = 80
NBUF = 4
EPT = NCHUNK_T * CHUNK
EP = NW * EPT
C2 = 48
ROWS_PER_TILE = NP // NS
BLK = 1024


def _mesh():
    return plsc.VectorSubcoreMesh(
        core_axis_name="c", subcore_axis_name="s", num_cores=NC, num_subcores=NS
    )


def _sc_deg(dst_p):
    def body(dst_hbm, out_hbm, dst_v, deg_v):
        c = lax.axis_index("c")
        s = lax.axis_index("s")
        w = s * NC + c
        zero16 = jnp.zeros((LANES,), jnp.float32)
        one16 = jnp.ones((LANES,), jnp.float32)

        def zloop(i, _):
            deg_v[pl.ds(i * LANES, LANES)] = zero16
            return None

        lax.fori_loop(0, NP // LANES, zloop, None)
        pltpu.sync_copy(dst_hbm.at[pl.ds(w * EPT, EPT)], dst_v)

        def eloop(i, _):
            idx = dst_v[pl.ds(i * LANES, LANES)]
            plsc.addupdate_scatter(deg_v, [idx], one16)
            return None

        lax.fori_loop(0, EPT // LANES, eloop, None)
        pltpu.sync_copy(deg_v, out_hbm.at[w])

    fn = pl.kernel(
        body,
        out_type=jax.ShapeDtypeStruct((NW, NP), jnp.float32),
        mesh=_mesh(),
        scratch_types=[
            pltpu.VMEM((EPT,), jnp.int32),
            pltpu.VMEM((NP,), jnp.float32),
        ],
        compiler_params=pltpu.CompilerParams(needs_layout_passes=False),
    )
    return fn(dst_p)


def _sc_scatter(g, src2, dst2, d):
    def body(g_hbm, src_hbm, dst_hbm, out_hbm, src_v, dst_v, rows, zbuf, acc, *sems):
        c = lax.axis_index("c")
        s = lax.axis_index("s")
        w = s * NC + c
        zero16 = jnp.zeros((LANES,), jnp.float32)

        def zrow(i, _):
            for j in range(d // LANES):
                zbuf[i, pl.ds(j * LANES, LANES)] = zero16
            return None

        lax.fori_loop(0, CHUNK, zrow, None)

        def zacc(k, _):
            pltpu.sync_copy(zbuf, acc.at[pl.ds(s * ROWS_PER_TILE + k * CHUNK, CHUNK)])
            return None

        lax.fori_loop(0, ROWS_PER_TILE // CHUNK, zacc, None)

        pltpu.sync_copy(src_hbm.at[pl.ds(w * NCHUNK_T, NCHUNK_T)], src_v)
        pltpu.sync_copy(dst_hbm.at[pl.ds(w * NCHUNK_T, NCHUNK_T)], dst_v)
        plsc.subcore_barrier()

        for b in range(NBUF):
            pltpu.async_copy(g_hbm.at[src_v.at[b]], rows.at[b], sems[b])

        def group(k, _):
            for b in range(NBUF):
                ch = k * NBUF + b
                pltpu.make_async_copy(g_hbm.at[src_v.at[ch]], rows.at[b], sems[b]).wait()
                pltpu.sync_copy(rows.at[b], acc.at[dst_v.at[ch]], add=True)

                @pl.when(ch + NBUF < NCHUNK_T)
                def _():
                    pltpu.async_copy(g_hbm.at[src_v.at[ch + NBUF]], rows.at[b], sems[b])

            return None

        lax.fori_loop(0, NCHUNK_T // NBUF, group, None)
        plsc.subcore_barrier()

        def flush(k, _):
            a = s * ROWS_PER_TILE + k * CHUNK
            pltpu.sync_copy(acc.at[pl.ds(a, CHUNK)], out_hbm.at[c, pl.ds(a, CHUNK)])
            return None

        lax.fori_loop(0, ROWS_PER_TILE // CHUNK, flush, None)

    fn = pl.kernel(
        body,
        out_type=jax.ShapeDtypeStruct((NC, NP, d), jnp.float32),
        mesh=_mesh(),
        scratch_types=[
            pltpu.VMEM((NCHUNK_T, CHUNK), jnp.int32),
            pltpu.VMEM((NCHUNK_T, CHUNK), jnp.int32),
            pltpu.VMEM((NBUF, CHUNK, d), jnp.float32),
            pltpu.VMEM((CHUNK, d), jnp.float32),
            pltpu.VMEM_SHARED((NP, d), jnp.float32),
        ] + [pltpu.SemaphoreType.DMA] * NBUF,
        compiler_params=pltpu.CompilerParams(
            needs_layout_passes=False, use_tc_tiling_on_sc=False
        ),
    )
    return fn(g, src2, dst2)


def _tc_pre_body(degp_ref, data_ref, w1_ref, h1_ref, g1_ref, dis_ref):
    deg = jnp.sum(degp_ref[...], axis=0) + 1.0
    dis = lax.rsqrt(deg)[:, None]
    h = jnp.dot(data_ref[...], w1_ref[...], preferred_element_type=jnp.float32)
    h1_ref[...] = h
    g1_ref[...] = h * dis
    dis_ref[...] = dis


def _tc_pre(degp, data_p, W1):
    grid = NP // BLK
    return pl.pallas_call(
        _tc_pre_body,
        grid=(grid,),
        in_specs=[
            pl.BlockSpec((NW, BLK), lambda i: (0, i)),
            pl.BlockSpec((BLK, N_FEAT), lambda i: (i, 0)),
            pl.BlockSpec((N_FEAT, HIDN), lambda i: (0, 0)),
        ],
        out_specs=[
            pl.BlockSpec((BLK, HIDN), lambda i: (i, 0)),
            pl.BlockSpec((BLK, HIDN), lambda i: (i, 0)),
            pl.BlockSpec((BLK, 1), lambda i: (i, 0)),
        ],
        out_shape=[
            jax.ShapeDtypeStruct((NP, HIDN), jnp.float32),
            jax.ShapeDtypeStruct((NP, HIDN), jnp.float32),
            jax.ShapeDtypeStruct((NP, 1), jnp.float32),
        ],
    )(degp, data_p, W1)


def _tc_mid_body(p1_ref, h1_ref, dis_ref, w2_ref, b1_ref, g2_ref, h2_ref):
    i = pl.program_id(0)
    dis = dis_ref[...]
    acc = p1_ref[0] + p1_ref[1]
    x1 = dis * acc + (dis * dis) * h1_ref[...] + b1_ref[...]
    x1 = jnp.maximum(x1, 0.0)
    rows = i * BLK + lax.broadcasted_iota(jnp.int32, (BLK, 1), 0)
    x1 = jnp.where(rows < N_NODES, x1, 0.0)
    h2 = jnp.dot(x1, w2_ref[...], preferred_element_type=jnp.float32)
    h2_ref[...] = h2
    g2_ref[...] = h2 * dis


def _tc_mid(p1, h1, dis, w2p, b1r):
    grid = NP // BLK
    return pl.pallas_call(
        _tc_mid_body,
        grid=(grid,),
        in_specs=[
            pl.BlockSpec((NC, BLK, HIDN), lambda i: (0, i, 0)),
            pl.BlockSpec((BLK, HIDN), lambda i: (i, 0)),
            pl.BlockSpec((BLK, 1), lambda i: (i, 0)),
            pl.BlockSpec((HIDN, C2), lambda i: (0, 0)),
            pl.BlockSpec((1, HIDN), lambda i: (0, 0)),
        ],
        out_specs=[
            pl.BlockSpec((BLK, C2), lambda i: (i, 0)),
            pl.BlockSpec((BLK, C2), lambda i: (i, 0)),
        ],
        out_shape=[
            jax.ShapeDtypeStruct((NP, C2), jnp.float32),
            jax.ShapeDtypeStruct((NP, C2), jnp.float32),
        ],
    )(p1, h1, dis, w2p, b1r)


def _tc_post_body(p2_ref, h2_ref, dis_ref, b2_ref, out_ref):
    dis = dis_ref[...]
    acc = p2_ref[0] + p2_ref[1]
    out_ref[...] = dis * acc + (dis * dis) * h2_ref[...] + b2_ref[...]


def _tc_post(p2, h2, dis, b2r):
    grid = NP // BLK
    return pl.pallas_call(
        _tc_post_body,
        grid=(grid,),
        in_specs=[
            pl.BlockSpec((NC, BLK, C2), lambda i: (0, i, 0)),
            pl.BlockSpec((BLK, C2), lambda i: (i, 0)),
            pl.BlockSpec((BLK, 1), lambda i: (i, 0)),
            pl.BlockSpec((1, C2), lambda i: (0, 0)),
        ],
        out_specs=pl.BlockSpec((BLK, C2), lambda i: (i, 0)),
        out_shape=jax.ShapeDtypeStruct((NP, C2), jnp.float32),
    )(p2, h2, dis, b2r)


def kernel(data, adj, W1, b1, W2, b2):
    src = adj[0].astype(jnp.int32)
    dst = adj[1].astype(jnp.int32)
    pad = jnp.full((EP - N_EDGES,), PAD, jnp.int32)
    src_p = jnp.concatenate([src, pad])
    dst_p = jnp.concatenate([dst, pad])
    src2 = src_p.reshape(EP // CHUNK, CHUNK)
    dst2 = dst_p.reshape(EP // CHUNK, CHUNK)
    data_p = jnp.zeros((NP, N_FEAT), jnp.float32).at[:N_NODES].set(data)
    w2p = jnp.zeros((HIDN, C2), jnp.float32).at[:, :N_CLASSES].set(W2)
    b1r = b1.reshape(1, HIDN)
    b2r = jnp.zeros((1, C2), jnp.float32).at[0, :N_CLASSES].set(b2)

    degp = _sc_deg(dst_p)
    h1, g1, dis = _tc_pre(degp, data_p, W1)
    p1 = _sc_scatter(g1, src2, dst2, HIDN)
    g2, h2 = _tc_mid(p1, h1, dis, w2p, b1r)
    p2 = _sc_scatter(g2, src2, dst2, C2)
    outp = _tc_post(p2, h2, dis, b2r)
    return outp[:N_NODES, :N_CLASSES]


# speedup vs baseline: 24.0538x; 1.0042x over previous
import functools

import jax
import jax.numpy as jnp
from jax import lax
from jax.experimental import pallas as pl
from jax.experimental.pallas import tpu as pltpu
from jax.experimental.pallas import tpu_sc as plsc

N_NODES = 10000
N_EDGES = 320000
N_FEAT = 128
HIDN = 16
N_CLASSES = 40

NP = 10240
PAD = N_NODES
NC, NS, LANES = 2, 16, 16
NW = NC * NS
CHUNK = 128
NCHUNK_T = 80
NBUF = 8
EPT = NCHUNK_T * CHUNK
EP = NW * EPT
C2 = 48
ROWS_PER_TILE = NP // NS
BLK = 1024


def _mesh():
    return plsc.VectorSubcoreMesh(
        core_axis_name="c", subcore_axis_name="s", num_cores=NC, num_subcores=NS
    )


def _sc_deg(dst_p):
    def body(dst_hbm, out_hbm, dst_v, deg_v):
        c = lax.axis_index("c")
        s = lax.axis_index("s")
        w = s * NC + c
        zero16 = jnp.zeros((LANES,), jnp.float32)
        one16 = jnp.ones((LANES,), jnp.float32)

        def zloop(i, _):
            deg_v[pl.ds(i * LANES, LANES)] = zero16
            return None

        lax.fori_loop(0, NP // LANES, zloop, None)
        pltpu.sync_copy(dst_hbm.at[pl.ds(w * EPT, EPT)], dst_v)

        def eloop(i, _):
            idx = dst_v[pl.ds(i * LANES, LANES)]
            plsc.addupdate_scatter(deg_v, [idx], one16)
            return None

        lax.fori_loop(0, EPT // LANES, eloop, None)
        pltpu.sync_copy(deg_v, out_hbm.at[w])

    fn = pl.kernel(
        body,
        out_type=jax.ShapeDtypeStruct((NW, NP), jnp.float32),
        mesh=_mesh(),
        scratch_types=[
            pltpu.VMEM((EPT,), jnp.int32),
            pltpu.VMEM((NP,), jnp.float32),
        ],
        compiler_params=pltpu.CompilerParams(needs_layout_passes=False),
    )
    return fn(dst_p)


def _sc_scatter(g, src2, dst2, d):
    def body(g_hbm, src_hbm, dst_hbm, out_hbm, src_v, dst_v, rows, zbuf, acc, *sems):
        c = lax.axis_index("c")
        s = lax.axis_index("s")
        w = s * NC + c
        zero16 = jnp.zeros((LANES,), jnp.float32)

        def zrow(i, _):
            for j in range(d // LANES):
                zbuf[i, pl.ds(j * LANES, LANES)] = zero16
            return None

        lax.fori_loop(0, CHUNK, zrow, None)

        def zacc(k, _):
            pltpu.sync_copy(zbuf, acc.at[pl.ds(s * ROWS_PER_TILE + k * CHUNK, CHUNK)])
            return None

        lax.fori_loop(0, ROWS_PER_TILE // CHUNK, zacc, None)

        pltpu.sync_copy(src_hbm.at[pl.ds(w * NCHUNK_T, NCHUNK_T)], src_v)
        pltpu.sync_copy(dst_hbm.at[pl.ds(w * NCHUNK_T, NCHUNK_T)], dst_v)
        plsc.subcore_barrier()

        for b in range(NBUF):
            pltpu.async_copy(g_hbm.at[src_v.at[b]], rows.at[b], sems[b])

        def group(k, _):
            for b in range(NBUF):
                ch = k * NBUF + b
                pltpu.make_async_copy(g_hbm.at[src_v.at[ch]], rows.at[b], sems[b]).wait()
                pltpu.sync_copy(rows.at[b], acc.at[dst_v.at[ch]], add=True)

                @pl.when(ch + NBUF < NCHUNK_T)
                def _():
                    pltpu.async_copy(g_hbm.at[src_v.at[ch + NBUF]], rows.at[b], sems[b])

            return None

        lax.fori_loop(0, NCHUNK_T // NBUF, group, None)
        plsc.subcore_barrier()

        def flush(k, _):
            a = s * ROWS_PER_TILE + k * CHUNK
            pltpu.sync_copy(acc.at[pl.ds(a, CHUNK)], out_hbm.at[c, pl.ds(a, CHUNK)])
            return None

        lax.fori_loop(0, ROWS_PER_TILE // CHUNK, flush, None)

    fn = pl.kernel(
        body,
        out_type=jax.ShapeDtypeStruct((NC, NP, d), jnp.float32),
        mesh=_mesh(),
        scratch_types=[
            pltpu.VMEM((NCHUNK_T, CHUNK), jnp.int32),
            pltpu.VMEM((NCHUNK_T, CHUNK), jnp.int32),
            pltpu.VMEM((NBUF, CHUNK, d), jnp.float32),
            pltpu.VMEM((CHUNK, d), jnp.float32),
            pltpu.VMEM_SHARED((NP, d), jnp.float32),
        ] + [pltpu.SemaphoreType.DMA] * NBUF,
        compiler_params=pltpu.CompilerParams(
            needs_layout_passes=False, use_tc_tiling_on_sc=False
        ),
    )
    return fn(g, src2, dst2)


def _tc_pre_body(degp_ref, data_ref, w1_ref, h1_ref, g1_ref, dis_ref):
    deg = jnp.sum(degp_ref[...], axis=0) + 1.0
    dis = lax.rsqrt(deg)[:, None]
    h = jnp.dot(data_ref[...], w1_ref[...], preferred_element_type=jnp.float32)
    h1_ref[...] = h
    g1_ref[...] = h * dis
    dis_ref[...] = dis


def _tc_pre(degp, data_p, W1):
    grid = NP // BLK
    return pl.pallas_call(
        _tc_pre_body,
        grid=(grid,),
        in_specs=[
            pl.BlockSpec((NW, BLK), lambda i: (0, i)),
            pl.BlockSpec((BLK, N_FEAT), lambda i: (i, 0)),
            pl.BlockSpec((N_FEAT, HIDN), lambda i: (0, 0)),
        ],
        out_specs=[
            pl.BlockSpec((BLK, HIDN), lambda i: (i, 0)),
            pl.BlockSpec((BLK, HIDN), lambda i: (i, 0)),
            pl.BlockSpec((BLK, 1), lambda i: (i, 0)),
        ],
        out_shape=[
            jax.ShapeDtypeStruct((NP, HIDN), jnp.float32),
            jax.ShapeDtypeStruct((NP, HIDN), jnp.float32),
            jax.ShapeDtypeStruct((NP, 1), jnp.float32),
        ],
    )(degp, data_p, W1)


def _tc_mid_body(p1_ref, h1_ref, dis_ref, w2_ref, b1_ref, g2_ref, h2_ref):
    i = pl.program_id(0)
    dis = dis_ref[...]
    acc = p1_ref[0] + p1_ref[1]
    x1 = dis * acc + (dis * dis) * h1_ref[...] + b1_ref[...]
    x1 = jnp.maximum(x1, 0.0)
    rows = i * BLK + lax.broadcasted_iota(jnp.int32, (BLK, 1), 0)
    x1 = jnp.where(rows < N_NODES, x1, 0.0)
    h2 = jnp.dot(x1, w2_ref[...], preferred_element_type=jnp.float32)
    h2_ref[...] = h2
    g2_ref[...] = h2 * dis


def _tc_mid(p1, h1, dis, w2p, b1r):
    grid = NP // BLK
    return pl.pallas_call(
        _tc_mid_body,
        grid=(grid,),
        in_specs=[
            pl.BlockSpec((NC, BLK, HIDN), lambda i: (0, i, 0)),
            pl.BlockSpec((BLK, HIDN), lambda i: (i, 0)),
            pl.BlockSpec((BLK, 1), lambda i: (i, 0)),
            pl.BlockSpec((HIDN, C2), lambda i: (0, 0)),
            pl.BlockSpec((1, HIDN), lambda i: (0, 0)),
        ],
        out_specs=[
            pl.BlockSpec((BLK, C2), lambda i: (i, 0)),
            pl.BlockSpec((BLK, C2), lambda i: (i, 0)),
        ],
        out_shape=[
            jax.ShapeDtypeStruct((NP, C2), jnp.float32),
            jax.ShapeDtypeStruct((NP, C2), jnp.float32),
        ],
    )(p1, h1, dis, w2p, b1r)


def _tc_post_body(p2_ref, h2_ref, dis_ref, b2_ref, out_ref):
    dis = dis_ref[...]
    acc = p2_ref[0] + p2_ref[1]
    out_ref[...] = dis * acc + (dis * dis) * h2_ref[...] + b2_ref[...]


def _tc_post(p2, h2, dis, b2r):
    grid = NP // BLK
    return pl.pallas_call(
        _tc_post_body,
        grid=(grid,),
        in_specs=[
            pl.BlockSpec((NC, BLK, C2), lambda i: (0, i, 0)),
            pl.BlockSpec((BLK, C2), lambda i: (i, 0)),
            pl.BlockSpec((BLK, 1), lambda i: (i, 0)),
            pl.BlockSpec((1, C2), lambda i: (0, 0)),
        ],
        out_specs=pl.BlockSpec((BLK, C2), lambda i: (i, 0)),
        out_shape=jax.ShapeDtypeStruct((NP, C2), jnp.float32),
    )(p2, h2, dis, b2r)


def kernel(data, adj, W1, b1, W2, b2):
    src = adj[0].astype(jnp.int32)
    dst = adj[1].astype(jnp.int32)
    pad = jnp.full((EP - N_EDGES,), PAD, jnp.int32)
    src_p = jnp.concatenate([src, pad])
    dst_p = jnp.concatenate([dst, pad])
    src2 = src_p.reshape(EP // CHUNK, CHUNK)
    dst2 = dst_p.reshape(EP // CHUNK, CHUNK)
    data_p = jnp.zeros((NP, N_FEAT), jnp.float32).at[:N_NODES].set(data)
    w2p = jnp.zeros((HIDN, C2), jnp.float32).at[:, :N_CLASSES].set(W2)
    b1r = b1.reshape(1, HIDN)
    b2r = jnp.zeros((1, C2), jnp.float32).at[0, :N_CLASSES].set(b2)

    degp = _sc_deg(dst_p)
    h1, g1, dis = _tc_pre(degp, data_p, W1)
    p1 = _sc_scatter(g1, src2, dst2, HIDN)
    g2, h2 = _tc_mid(p1, h1, dis, w2p, b1r)
    p2 = _sc_scatter(g2, src2, dst2, C2)
    outp = _tc_post(p2, h2, dis, b2r)
    return outp[:N_NODES, :N_CLASSES]
